# Initial kernel scaffold; baseline (speedup 1.0000x reference)
import jax
import jax.numpy as jnp
from jax.experimental import pallas as pl


def kernel(x, W_gate, up_proj, gate_proj, down_proj):
    raise NotImplementedError("write your pallas kernel here")


# speedup vs baseline: 24.6168x; 24.6168x over previous
import functools

import jax
import jax.numpy as jnp
from jax.experimental import pallas as pl
from jax.experimental.pallas import tpu as pltpu


def _moe_body(x_ref, wg_ref, up_ref, gate_ref, down_ref, out_ref, w_scr):
    e = pl.program_id(0)

    @pl.when(e == 0)
    def _compute_gating():
        xf = x_ref[...].astype(jnp.float32)
        logits = jnp.dot(xf, wg_ref[...], preferred_element_type=jnp.float32)
        m = jnp.max(logits, axis=-1, keepdims=True)
        p = jnp.exp(logits - m)
        g = p / jnp.sum(p, axis=-1, keepdims=True)
        col = jax.lax.broadcasted_iota(jnp.int32, g.shape, 1)
        i1 = jnp.argmax(g, axis=-1)[:, None]
        oh1 = col == i1
        i2 = jnp.argmax(jnp.where(oh1, -1.0, g), axis=-1)[:, None]
        oh2 = col == i2
        w_scr[...] = jnp.where(oh1 | oh2, g, 0.0)

    xb = x_ref[...]
    up_w = up_ref[0]
    gate_w = gate_ref[0]
    down_w = down_ref[0]
    dn = (((1,), (1,)), ((), ()))
    up = jax.lax.dot_general(xb, up_w, dn, preferred_element_type=jnp.float32)
    gate = jax.lax.dot_general(xb, gate_w, dn, preferred_element_type=jnp.float32)
    hidden = (gate * jax.nn.sigmoid(gate) * up).astype(jnp.bfloat16)
    y = jax.lax.dot_general(hidden, down_w, dn, preferred_element_type=jnp.float32)
    wcol = jax.lax.broadcasted_iota(jnp.int32, w_scr.shape, 1)
    we = jnp.sum(jnp.where(wcol == e, w_scr[...], 0.0), axis=1, keepdims=True)
    contrib = we * y

    @pl.when(e == 0)
    def _init():
        out_ref[...] = contrib

    @pl.when(e > 0)
    def _acc():
        out_ref[...] += contrib


@jax.jit
def kernel(x, W_gate, up_proj, gate_proj, down_proj):
    S, D = x.shape
    E, H, _ = up_proj.shape
    return pl.pallas_call(
        _moe_body,
        grid=(E,),
        in_specs=[
            pl.BlockSpec((S, D), lambda e: (0, 0)),
            pl.BlockSpec((D, E), lambda e: (0, 0)),
            pl.BlockSpec((1, H, D), lambda e: (e, 0, 0)),
            pl.BlockSpec((1, H, D), lambda e: (e, 0, 0)),
            pl.BlockSpec((1, D, H), lambda e: (e, 0, 0)),
        ],
        out_specs=pl.BlockSpec((S, D), lambda e: (0, 0)),
        out_shape=jax.ShapeDtypeStruct((S, D), jnp.float32),
        scratch_shapes=[pltpu.VMEM((S, E), jnp.float32)],
    )(x, W_gate, up_proj, gate_proj, down_proj)
